# Initial kernel scaffold; baseline (speedup 1.0000x reference)
import jax
import jax.numpy as jnp
from jax.experimental import pallas as pl


def kernel(x, adj, W1, b1, W2, b2):
    raise NotImplementedError("write your pallas kernel here")


# speedup vs baseline: 15.1525x; 15.1525x over previous
import functools

import jax
import jax.numpy as jnp
from jax import lax
from jax.experimental import pallas as pl
from jax.experimental.pallas import tpu as pltpu
from jax.experimental.pallas import tpu_sc as plsc

NC, NS, LANES = 2, 16, 16
NW = NC * NS

N = 10000
E = 320000
NFEAT = 128
NHID = 128
NCLASS = 16

EPW = E // NW
K = 80
NCH = EPW // K
N_PAD = 10240
ROWS_T = N_PAD // NS

@functools.cache
def _mesh():
    return plsc.VectorSubcoreMesh(core_axis_name="c", subcore_axis_name="s",
                                  num_cores=NC, num_subcores=NS)


def _degree_body(src_hbm, dst_hbm, zi_hbm, out_hbm, src_v, dst_v, hs_v, hd_v):
    c = lax.axis_index("c")
    s = lax.axis_index("s")

    pltpu.sync_copy(src_hbm.at[c, s], src_v)
    pltpu.sync_copy(dst_hbm.at[c, s], dst_v)
    pltpu.sync_copy(zi_hbm, hs_v)
    pltpu.sync_copy(zi_hbm, hd_v)

    ones = jnp.ones((LANES,), jnp.int32)

    def body(i, carry):
        si = src_v[pl.ds(i * LANES, LANES)]
        plsc.addupdate_scatter(hs_v, [si], ones)
        di = dst_v[pl.ds(i * LANES, LANES)]
        plsc.addupdate_scatter(hd_v, [di], ones)
        return carry

    lax.fori_loop(0, EPW // LANES, body, 0)

    pltpu.sync_copy(hs_v, out_hbm.at[c, s, 0])
    pltpu.sync_copy(hd_v, out_hbm.at[c, s, 1])


@functools.cache
def _degree_call():
    return pl.kernel(
        _degree_body,
        out_type=jax.ShapeDtypeStruct((NC, NS, 2, N), jnp.int32),
        mesh=_mesh(),
        compiler_params=pltpu.CompilerParams(needs_layout_passes=False),
        scratch_types=[
            pltpu.VMEM((EPW,), jnp.int32),
            pltpu.VMEM((EPW,), jnp.int32),
            pltpu.VMEM((N,), jnp.int32),
            pltpu.VMEM((N,), jnp.int32),
        ],
    )


def _agg_body(d, table_hbm, src_hbm, dst_hbm, z_hbm, out_hbm,
              src_v, dst_v, rows_v, gsem, acc_sh):
    c = lax.axis_index("c")
    s = lax.axis_index("s")

    pltpu.sync_copy(z_hbm.at[pl.ds(s * ROWS_T, ROWS_T)],
                    acc_sh.at[pl.ds(s * ROWS_T, ROWS_T)])
    pltpu.sync_copy(src_hbm.at[c, s], src_v)
    pltpu.sync_copy(dst_hbm.at[c, s], dst_v)
    plsc.subcore_barrier()

    def body(j, carry):
        pltpu.async_copy(table_hbm.at[src_v.at[pl.ds(j * K, K)]],
                         rows_v, gsem).wait()
        pltpu.sync_copy(rows_v, acc_sh.at[dst_v.at[j]], add=True)
        return carry

    lax.fori_loop(0, NCH, body, 0)

    plsc.subcore_barrier()
    pltpu.sync_copy(acc_sh.at[pl.ds(s * ROWS_T, ROWS_T)],
                    out_hbm.at[c, pl.ds(s * ROWS_T, ROWS_T)])


@functools.cache
def _make_agg(d):
    return pl.kernel(
        functools.partial(_agg_body, d),
        out_type=jax.ShapeDtypeStruct((NC, N_PAD, d), jnp.float32),
        mesh=_mesh(),
        compiler_params=pltpu.CompilerParams(
            use_tc_tiling_on_sc=(d % 128 == 0)),
        scratch_types=[
            pltpu.VMEM((EPW,), jnp.int32),
            pltpu.VMEM((NCH, K), jnp.int32),
            pltpu.VMEM((K, d), jnp.float32),
            pltpu.SemaphoreType.DMA,
            pltpu.VMEM_SHARED((N_PAD, d), jnp.float32),
        ],
    )


_RB = 1000


def _support1_body(x_ref, w_ref, hist_ref, o_ref, deg_ref):
    d = jnp.sum(hist_ref[...], axis=0)
    deg_ref[...] = d
    a = lax.rsqrt(jnp.maximum(d[:, 0:1].astype(jnp.float32), 1.0))
    o_ref[...] = jnp.dot(x_ref[...], w_ref[...],
                         preferred_element_type=jnp.float32) * a


def _support1_call(x, w1, hist):
    return pl.pallas_call(
        _support1_body,
        grid=(N // _RB,),
        in_specs=[
            pl.BlockSpec((_RB, NFEAT), lambda i: (i, 0)),
            pl.BlockSpec((NFEAT, NHID), lambda i: (0, 0)),
            pl.BlockSpec((NW, _RB, 2), lambda i: (0, i, 0)),
        ],
        out_specs=[
            pl.BlockSpec((_RB, NHID), lambda i: (i, 0)),
            pl.BlockSpec((_RB, 2), lambda i: (i, 0)),
        ],
        out_shape=[
            jax.ShapeDtypeStruct((N, NHID), jnp.float32),
            jax.ShapeDtypeStruct((N, 2), jnp.int32),
        ],
    )(x, w1, hist)


def _layer2_body(p_ref, deg_ref, b1_ref, w2_ref, o_ref):
    agg = p_ref[0] + p_ref[1]
    d = deg_ref[...].astype(jnp.float32)
    bsc = lax.rsqrt(jnp.maximum(d[:, 1:2], 1.0))
    h = jnp.maximum(agg * bsc + b1_ref[...], 0.0)
    a = lax.rsqrt(jnp.maximum(d[:, 0:1], 1.0))
    o_ref[...] = jnp.dot(h, w2_ref[...],
                         preferred_element_type=jnp.float32) * a


def _layer2_call(p1, deg2, b1, w2):
    return pl.pallas_call(
        _layer2_body,
        grid=(N // _RB,),
        in_specs=[
            pl.BlockSpec((NC, _RB, NHID), lambda i: (0, i, 0)),
            pl.BlockSpec((_RB, 2), lambda i: (i, 0)),
            pl.BlockSpec((1, NHID), lambda i: (0, 0)),
            pl.BlockSpec((NHID, NCLASS), lambda i: (0, 0)),
        ],
        out_specs=pl.BlockSpec((_RB, NCLASS), lambda i: (i, 0)),
        out_shape=jax.ShapeDtypeStruct((N, NCLASS), jnp.float32),
    )(p1, deg2, b1, w2)


def _final_body(p_ref, deg_ref, b2_ref, o_ref):
    d = deg_ref[...].astype(jnp.float32)
    bsc = lax.rsqrt(jnp.maximum(d[:, 1:2], 1.0))
    z = (p_ref[0] + p_ref[1]) * bsc + b2_ref[...]
    m = jnp.max(z, axis=1, keepdims=True)
    ez = jnp.exp(z - m)
    lse = jnp.log(jnp.sum(ez, axis=1, keepdims=True)) + m
    o_ref[...] = z - lse


def _final_call(p2, deg2, b2):
    return pl.pallas_call(
        _final_body,
        grid=(N // _RB,),
        in_specs=[
            pl.BlockSpec((NC, _RB, NCLASS), lambda i: (0, i, 0)),
            pl.BlockSpec((_RB, 2), lambda i: (i, 0)),
            pl.BlockSpec((1, NCLASS), lambda i: (0, 0)),
        ],
        out_specs=pl.BlockSpec((_RB, NCLASS), lambda i: (i, 0)),
        out_shape=jax.ShapeDtypeStruct((N, NCLASS), jnp.float32),
    )(p2, deg2, b2)


def kernel(x, adj, W1, b1, W2, b2):
    src = adj[0].reshape(NC, NS, EPW)
    dst_flat = adj[1].reshape(NC, NS, EPW)
    dst_chunks = adj[1].reshape(NC, NS, NCH, K)
    zi = jnp.zeros((N,), jnp.int32)
    z128 = jnp.zeros((N_PAD, NHID), jnp.float32)
    z16 = jnp.zeros((N_PAD, NCLASS), jnp.float32)

    degp = _degree_call()(src, dst_flat, zi)
    hist = degp.reshape(NW, 2, N).transpose(0, 2, 1)

    s1, deg2 = _support1_call(x, W1, hist)
    p1 = _make_agg(NHID)(s1, src, dst_chunks, z128)
    s2 = _layer2_call(p1, deg2, b1.reshape(1, NHID), W2)
    p2 = _make_agg(NCLASS)(s2, src, dst_chunks, z16)
    return _final_call(p2, deg2, b2.reshape(1, NCLASS))


# speedup vs baseline: 20.2616x; 1.3372x over previous
import functools

import jax
import jax.numpy as jnp
from jax import lax
from jax.experimental import pallas as pl
from jax.experimental.pallas import tpu as pltpu
from jax.experimental.pallas import tpu_sc as plsc

NC, NS, LANES = 2, 16, 16
NW = NC * NS

N = 10000
E = 320000
NFEAT = 128
NHID = 128
NCLASS = 16

EPW = E // NW
K = 80
NCH = EPW // K
N_PAD = 10240
ROWS_T = N_PAD // NS

@functools.cache
def _mesh():
    return plsc.VectorSubcoreMesh(core_axis_name="c", subcore_axis_name="s",
                                  num_cores=NC, num_subcores=NS)


def _degree_body(src_hbm, dst_hbm, zi_hbm, out_hbm, src_v, dst_v, hs_v, hd_v):
    c = lax.axis_index("c")
    s = lax.axis_index("s")

    pltpu.sync_copy(src_hbm.at[c, s], src_v)
    pltpu.sync_copy(dst_hbm.at[c, s], dst_v)
    pltpu.sync_copy(zi_hbm, hs_v)
    pltpu.sync_copy(zi_hbm, hd_v)

    ones = jnp.ones((LANES,), jnp.int32)

    def body(i, carry):
        si = src_v[pl.ds(i * LANES, LANES)]
        plsc.addupdate_scatter(hs_v, [si], ones)
        di = dst_v[pl.ds(i * LANES, LANES)]
        plsc.addupdate_scatter(hd_v, [di], ones)
        return carry

    lax.fori_loop(0, EPW // LANES, body, 0)

    pltpu.sync_copy(hs_v, out_hbm.at[c, s, 0])
    pltpu.sync_copy(hd_v, out_hbm.at[c, s, 1])


@functools.cache
def _degree_call():
    return pl.kernel(
        _degree_body,
        out_type=jax.ShapeDtypeStruct((NC, NS, 2, N), jnp.int32),
        mesh=_mesh(),
        compiler_params=pltpu.CompilerParams(needs_layout_passes=False),
        scratch_types=[
            pltpu.VMEM((EPW,), jnp.int32),
            pltpu.VMEM((EPW,), jnp.int32),
            pltpu.VMEM((N,), jnp.int32),
            pltpu.VMEM((N,), jnp.int32),
        ],
    )


def _agg_body(d, table_hbm, src_hbm, dst_hbm, z_hbm, out_hbm,
              src_v, dst_v, rows0_v, rows1_v, sem0, sem1, acc_sh):
    c = lax.axis_index("c")
    s = lax.axis_index("s")

    pltpu.sync_copy(z_hbm.at[pl.ds(s * ROWS_T, ROWS_T)],
                    acc_sh.at[pl.ds(s * ROWS_T, ROWS_T)])
    pltpu.sync_copy(src_hbm.at[c, s], src_v)
    pltpu.sync_copy(dst_hbm.at[c, s], dst_v)
    plsc.subcore_barrier()

    def gather(chunk, rows, sem):
        return pltpu.async_copy(
            table_hbm.at[src_v.at[pl.ds(chunk * K, K)]], rows, sem)

    gather(0, rows0_v, sem0)

    def body(jj, carry):
        c0 = 2 * jj
        gather(c0 + 1, rows1_v, sem1)
        pltpu.make_async_copy(table_hbm.at[src_v.at[pl.ds(0, K)]],
                              rows0_v, sem0).wait()
        pltpu.sync_copy(rows0_v, acc_sh.at[dst_v.at[c0]], add=True)
        gather(c0 + 2, rows0_v, sem0)
        pltpu.make_async_copy(table_hbm.at[src_v.at[pl.ds(0, K)]],
                              rows1_v, sem1).wait()
        pltpu.sync_copy(rows1_v, acc_sh.at[dst_v.at[c0 + 1]], add=True)
        return carry

    lax.fori_loop(0, (NCH - 1) // 2, body, 0)
    pltpu.make_async_copy(table_hbm.at[src_v.at[pl.ds(0, K)]],
                          rows0_v, sem0).wait()
    pltpu.sync_copy(rows0_v, acc_sh.at[dst_v.at[NCH - 1]], add=True)

    plsc.subcore_barrier()
    pltpu.sync_copy(acc_sh.at[pl.ds(s * ROWS_T, ROWS_T)],
                    out_hbm.at[c, pl.ds(s * ROWS_T, ROWS_T)])


@functools.cache
def _make_agg(d):
    return pl.kernel(
        functools.partial(_agg_body, d),
        out_type=jax.ShapeDtypeStruct((NC, N_PAD, d), jnp.float32),
        mesh=_mesh(),
        compiler_params=pltpu.CompilerParams(
            use_tc_tiling_on_sc=(d % 128 == 0)),
        scratch_types=[
            pltpu.VMEM((EPW,), jnp.int32),
            pltpu.VMEM((NCH, K), jnp.int32),
            pltpu.VMEM((K, d), jnp.float32),
            pltpu.VMEM((K, d), jnp.float32),
            pltpu.SemaphoreType.DMA,
            pltpu.SemaphoreType.DMA,
            pltpu.VMEM_SHARED((N_PAD, d), jnp.float32),
        ],
    )


_RB = 1000


def _support1_body(x_ref, w_ref, hist_ref, o_ref, deg_ref):
    d = jnp.sum(hist_ref[...], axis=0)
    deg_ref[...] = d
    a = lax.rsqrt(jnp.maximum(d[:, 0:1].astype(jnp.float32), 1.0))
    o_ref[...] = jnp.dot(x_ref[...], w_ref[...],
                         preferred_element_type=jnp.float32) * a


def _support1_call(x, w1, hist):
    return pl.pallas_call(
        _support1_body,
        grid=(N // _RB,),
        in_specs=[
            pl.BlockSpec((_RB, NFEAT), lambda i: (i, 0)),
            pl.BlockSpec((NFEAT, NHID), lambda i: (0, 0)),
            pl.BlockSpec((NW, _RB, 2), lambda i: (0, i, 0)),
        ],
        out_specs=[
            pl.BlockSpec((_RB, NHID), lambda i: (i, 0)),
            pl.BlockSpec((_RB, 2), lambda i: (i, 0)),
        ],
        out_shape=[
            jax.ShapeDtypeStruct((N, NHID), jnp.float32),
            jax.ShapeDtypeStruct((N, 2), jnp.int32),
        ],
    )(x, w1, hist)


def _layer2_body(p_ref, deg_ref, b1_ref, w2_ref, o_ref):
    agg = p_ref[0] + p_ref[1]
    d = deg_ref[...].astype(jnp.float32)
    bsc = lax.rsqrt(jnp.maximum(d[:, 1:2], 1.0))
    h = jnp.maximum(agg * bsc + b1_ref[...], 0.0)
    a = lax.rsqrt(jnp.maximum(d[:, 0:1], 1.0))
    o_ref[...] = jnp.dot(h, w2_ref[...],
                         preferred_element_type=jnp.float32) * a


def _layer2_call(p1, deg2, b1, w2):
    return pl.pallas_call(
        _layer2_body,
        grid=(N // _RB,),
        in_specs=[
            pl.BlockSpec((NC, _RB, NHID), lambda i: (0, i, 0)),
            pl.BlockSpec((_RB, 2), lambda i: (i, 0)),
            pl.BlockSpec((1, NHID), lambda i: (0, 0)),
            pl.BlockSpec((NHID, NCLASS), lambda i: (0, 0)),
        ],
        out_specs=pl.BlockSpec((_RB, NCLASS), lambda i: (i, 0)),
        out_shape=jax.ShapeDtypeStruct((N, NCLASS), jnp.float32),
    )(p1, deg2, b1, w2)


def _final_body(p_ref, deg_ref, b2_ref, o_ref):
    d = deg_ref[...].astype(jnp.float32)
    bsc = lax.rsqrt(jnp.maximum(d[:, 1:2], 1.0))
    z = (p_ref[0] + p_ref[1]) * bsc + b2_ref[...]
    m = jnp.max(z, axis=1, keepdims=True)
    ez = jnp.exp(z - m)
    lse = jnp.log(jnp.sum(ez, axis=1, keepdims=True)) + m
    o_ref[...] = z - lse


def _final_call(p2, deg2, b2):
    return pl.pallas_call(
        _final_body,
        grid=(N // _RB,),
        in_specs=[
            pl.BlockSpec((NC, _RB, NCLASS), lambda i: (0, i, 0)),
            pl.BlockSpec((_RB, 2), lambda i: (i, 0)),
            pl.BlockSpec((1, NCLASS), lambda i: (0, 0)),
        ],
        out_specs=pl.BlockSpec((_RB, NCLASS), lambda i: (i, 0)),
        out_shape=jax.ShapeDtypeStruct((N, NCLASS), jnp.float32),
    )(p2, deg2, b2)


def kernel(x, adj, W1, b1, W2, b2):
    src = adj[0].reshape(NC, NS, EPW)
    dst = adj[1].reshape(NC, NS, EPW)
    dst_chunks = dst.reshape(NC, NS, NCH, K)
    zi = jnp.zeros((N,), jnp.int32)
    z128 = jnp.zeros((N_PAD, NHID), jnp.float32)
    z16 = jnp.zeros((N_PAD, NCLASS), jnp.float32)

    degp = _degree_call()(src, dst, zi)
    hist = degp.reshape(NW, 2, N).transpose(0, 2, 1)

    s1, deg2 = _support1_call(x, W1, hist)
    p1 = _make_agg(NHID)(s1, src, dst_chunks, z128)
    s2 = _layer2_call(p1, deg2, b1.reshape(1, NHID), W2)
    p2 = _make_agg(NCLASS)(s2, src, dst_chunks, z16)
    return _final_call(p2, deg2, b2.reshape(1, NCLASS))
